# baseline (speedup 1.0000x reference)
import functools

import jax
import jax.numpy as jnp
from jax import lax
from jax.experimental import pallas as pl
from jax.experimental.pallas import tpu as pltpu
from jax.experimental.pallas import tpu_sc as plsc

F32 = jnp.float32
I32 = jnp.int32

_N = 10000
_E = 160000
_D = 256
_DE = 16
_L = 3
_H = 128
_NPAD = 10240
_EPAD = 163840
_CH = 128
_EPS = _EPAD // 16
_NCHUNK = _EPS // _CH
_STRIPE = _NPAD // 16


def _mm_bias_body(x_ref, w_ref, b_ref, o_ref):
    o_ref[...] = jnp.dot(x_ref[...], w_ref[...],
                         preferred_element_type=F32) + b_ref[...]


def _mm_bias(x, w, b, bn):
    n, k = x.shape
    m = w.shape[1]
    return pl.pallas_call(
        _mm_bias_body,
        grid=(n // bn,),
        in_specs=[pl.BlockSpec((bn, k), lambda i: (i, 0)),
                  pl.BlockSpec((k, m), lambda i: (0, 0)),
                  pl.BlockSpec((1, m), lambda i: (0, 0))],
        out_specs=pl.BlockSpec((bn, m), lambda i: (i, 0)),
        out_shape=jax.ShapeDtypeStruct((n, m), F32),
    )(x, w, b)


def _mm_split_scale_body(x_ref, w_ref, b_ref, sc_ref, o0_ref, o1_ref):
    z = (jnp.dot(x_ref[...], w_ref[...], preferred_element_type=F32)
         + b_ref[...]) * sc_ref[...]
    o0_ref[...] = z[:, :_H]
    o1_ref[...] = z[:, _H:]


def _mm_split_scale(x, w, b, sc, bn):
    n, k = x.shape
    return pl.pallas_call(
        _mm_split_scale_body,
        grid=(n // bn,),
        in_specs=[pl.BlockSpec((bn, k), lambda i: (i, 0)),
                  pl.BlockSpec((k, _D), lambda i: (0, 0)),
                  pl.BlockSpec((1, _D), lambda i: (0, 0)),
                  pl.BlockSpec((bn, 1), lambda i: (i, 0))],
        out_specs=[pl.BlockSpec((bn, _H), lambda i: (i, 0)),
                   pl.BlockSpec((bn, _H), lambda i: (i, 0))],
        out_shape=[jax.ShapeDtypeStruct((n, _H), F32),
                   jax.ShapeDtypeStruct((n, _H), F32)],
    )(x, w, b, sc)


def _mm_split4_body(x_ref, w_ref, b_ref, sc_ref, o0_ref, o1_ref,
                    s0_ref, s1_ref):
    z = jnp.dot(x_ref[...], w_ref[...], preferred_element_type=F32) + b_ref[...]
    o0_ref[...] = z[:, :_H]
    o1_ref[...] = z[:, _H:]
    zs = z * sc_ref[...]
    s0_ref[...] = zs[:, :_H]
    s1_ref[...] = zs[:, _H:]


def _mm_split4(x, w, b, sc, bn):
    n, k = x.shape
    half = pl.BlockSpec((bn, _H), lambda i: (i, 0))
    return pl.pallas_call(
        _mm_split4_body,
        grid=(n // bn,),
        in_specs=[pl.BlockSpec((bn, k), lambda i: (i, 0)),
                  pl.BlockSpec((k, _D), lambda i: (0, 0)),
                  pl.BlockSpec((1, _D), lambda i: (0, 0)),
                  pl.BlockSpec((bn, 1), lambda i: (i, 0))],
        out_specs=[half, half, half, half],
        out_shape=[jax.ShapeDtypeStruct((n, _H), F32)] * 4,
    )(x, w, b, sc)


def _post_body(a0, a1, h0, h1, invd, disn, root, gam, bet, mu, var, o_ref, *,
               relu_out):
    iv = invd[...]
    dv = disn[...]
    for half, (a, hh) in enumerate(((a0, h0), (a1, h1))):
        sl = slice(half * _H, (half + 1) * _H)
        t = a[...] * dv + jnp.maximum(hh[...] + root[:, sl], 0.0) * iv
        t = (t - mu[:, sl]) / jnp.sqrt(var[:, sl] + 1e-5) * gam[:, sl] \
            + bet[:, sl]
        if relu_out:
            t = jnp.maximum(t, 0.0)
        o_ref[:, sl] = t


def _post(a0, a1, h0, h1, invd, disn, root, gam, bet, mu, var, relu_out, bn):
    n = _N
    body = functools.partial(_post_body, relu_out=relu_out)
    half_spec = pl.BlockSpec((bn, _H), lambda i: (i, 0))
    col_spec = pl.BlockSpec((bn, 1), lambda i: (i, 0))
    par_spec = pl.BlockSpec((1, _D), lambda i: (0, 0))
    return pl.pallas_call(
        body,
        grid=(n // bn,),
        in_specs=[half_spec, half_spec, half_spec, half_spec,
                  col_spec, col_spec,
                  par_spec, par_spec, par_spec, par_spec, par_spec],
        out_specs=pl.BlockSpec((bn, _D), lambda i: (i, 0)),
        out_shape=jax.ShapeDtypeStruct((n, _D), F32),
    )(a0, a1, h0, h1, invd, disn, root, gam, bet, mu, var)


def _mesh():
    return plsc.VectorSubcoreMesh(core_axis_name="c", subcore_axis_name="s",
                                  num_cores=2, num_subcores=16)


def _degnorm_body(rowc_hbm, rowg_hbm, srow_hbm, dis_hbm, invd_hbm,
                  idxv, onesv, rowv, degv, disv, srowv, invv, deg_sp):
    c = lax.axis_index("c")
    s = lax.axis_index("s")

    def fill_inv(k, _):
        invv[pl.ds(k * 16, 16)] = jnp.full((16,), 1.0, F32)
        return 0
    lax.fori_loop(0, 40, fill_inv, 0)
    pltpu.sync_copy(invv, deg_sp.at[pl.ds(s * 640, 640)])

    def fill_ones(k, _):
        onesv[pl.ds(k * 16, 16)] = jnp.full((16,), 1.0, F32)
        return 0
    lax.fori_loop(0, 8, fill_ones, 0)
    plsc.subcore_barrier()

    def cnt(j, _):
        base = s * _EPS + j * _CH
        pltpu.sync_copy(rowc_hbm.at[pl.ds(base, _CH)], idxv)
        pltpu.sync_copy(onesv, deg_sp.at[idxv], add=True)
        return 0
    lax.fori_loop(0, _NCHUNK, cnt, 0)
    plsc.subcore_barrier()

    pltpu.sync_copy(deg_sp, degv)
    magic = jnp.full((16,), 0x5F3759DF, I32)

    def rsq(k, _):
        d = degv[pl.ds(k * 16, 16)]
        i = plsc.bitcast(d, I32)
        y = plsc.bitcast(magic - (i >> 1), F32)
        hd = 0.5 * d
        y = y * (1.5 - hd * y * y)
        y = y * (1.5 - hd * y * y)
        y = y * (1.5 - hd * y * y)
        disv[pl.ds(k * 16, 16)] = y
        return 0
    lax.fori_loop(0, _NPAD // 16, rsq, 0)

    @pl.when(c == 0)
    def _():
        pltpu.sync_copy(disv.at[pl.ds(s * 640, 640)],
                        dis_hbm.at[pl.ds(s * 640, 640)])

    @pl.when(c == 1)
    def _():
        def invf(k, _):
            d = degv[pl.ds(s * 640 + k * 16, 16)]
            invv[pl.ds(k * 16, 16)] = 1.0 / d
            return 0
        lax.fori_loop(0, 40, invf, 0)
        pltpu.sync_copy(invv, invd_hbm.at[pl.ds(s * 640, 640)])

    half = _EPS // 2
    ebase = s * _EPS + c * half
    pltpu.sync_copy(rowg_hbm.at[pl.ds(ebase, half)], rowv)

    def srw(k, _):
        ri = rowv[pl.ds(k * 16, 16)]
        srowv[pl.ds(k * 16, 16)] = plsc.load_gather(disv, [ri])
        return 0
    lax.fori_loop(0, half // 16, srw, 0)
    pltpu.sync_copy(srowv, srow_hbm.at[pl.ds(ebase, half)])


def _degnorm(rowc, rowg):
    f = pl.kernel(
        _degnorm_body,
        out_type=[jax.ShapeDtypeStruct((_EPAD,), F32),
                  jax.ShapeDtypeStruct((_NPAD,), F32),
                  jax.ShapeDtypeStruct((_NPAD,), F32)],
        mesh=_mesh(),
        scratch_types=[
            pltpu.VMEM((_CH,), I32),
            pltpu.VMEM((_CH,), F32),
            pltpu.VMEM((_EPS // 2,), I32),
            pltpu.VMEM((_NPAD,), F32),
            pltpu.VMEM((_NPAD,), F32),
            pltpu.VMEM((_EPS // 2,), F32),
            pltpu.VMEM((640,), F32),
            pltpu.VMEM_SHARED((_NPAD,), F32),
        ],
        compiler_params=pltpu.CompilerParams(needs_layout_passes=False),
    )
    return f(rowc, rowg)


_K = 4
_S = 2


def _aggr_body(hx0, hx1, ee0, ee1, rowg, colg, out0, out1,
               idxb, gb, agg_sp,
               sg0, sg1, sg2, sg3, se0, se1, se2, se3, ss0, ss1, ss2, ss3):
    c = lax.axis_index("c")
    s = lax.axis_index("s")
    sem_g = (sg0, sg1, sg2, sg3)
    sem_e = (se0, se1, se2, se3)
    sem_s = (ss0, ss1, ss2, ss3)

    def z(t, _):
        gb[t // 8, pl.ds((t % 8) * 16, 16)] = jnp.zeros((16,), F32)
        return 0
    lax.fori_loop(0, _S * _CH * 8, z, 0)
    nz = _S * _CH
    for q in range(0, _STRIPE, nz):
        rows = min(nz, _STRIPE - q)
        pltpu.sync_copy(gb.at[pl.ds(0, rows)],
                        agg_sp.at[pl.ds(s * _STRIPE + q, rows)])
    plsc.subcore_barrier()

    def run(hx, ee, out):
        def body(t, _):
            base = s * _EPS + t * (_K * _CH)
            pltpu.sync_copy(rowg.at[pl.ds(base, _K * _CH)], idxb.at[0])
            pltpu.sync_copy(colg.at[pl.ds(base, _K * _CH)], idxb.at[1])
            cg, ce, sc = {}, {}, {}

            def issue_ee(k):
                es = k % _S
                ce[k] = pltpu.async_copy(
                    ee.at[pl.ds(base + k * _CH, _CH)],
                    gb.at[pl.ds(es * _CH, _CH)], sem_e[es])

            def issue_gather(k):
                gs = k % _S
                cg[k] = pltpu.async_copy(
                    hx.at[idxb.at[0, pl.ds(k * _CH, _CH)]],
                    gb.at[pl.ds(gs * _CH, _CH)], sem_g[gs], add=True)

            for k in range(_S):
                issue_ee(k)
            ce[0].wait()
            issue_gather(0)
            for k in range(_K):
                gs = k % _S
                go = gs * _CH
                if k + 1 < _K:
                    ce[k + 1].wait()
                    issue_gather(k + 1)
                cg[k].wait()

                def group(g, _, go=go):
                    for j in range(8):
                        i = g * 8 + j
                        for r in range(_H // 16):
                            v = gb[go + i, pl.ds(r * 16, 16)]
                            gb[go + i, pl.ds(r * 16, 16)] = \
                                jnp.maximum(v, 0.0)
                    return 0
                lax.fori_loop(0, _CH // 8, group, 0)

                sc[k] = pltpu.async_copy(
                    gb.at[pl.ds(go, _CH)],
                    agg_sp.at[idxb.at[1, pl.ds(k * _CH, _CH)]], sem_s[gs],
                    add=True)
                if k + _S < _K:
                    sc[k].wait()
                    issue_ee(k + _S)
            for k in range(_K - _S, _K):
                sc[k].wait()
            return 0
        lax.fori_loop(0, _NCHUNK // _K, body, 0)
        plsc.subcore_barrier()
        pltpu.sync_copy(agg_sp.at[pl.ds(s * _STRIPE, _STRIPE)],
                        out.at[pl.ds(s * _STRIPE, _STRIPE)])

    @pl.when(c == 0)
    def _():
        run(hx0, ee0, out0)

    @pl.when(c == 1)
    def _():
        run(hx1, ee1, out1)


def _aggr(hx0, hx1, ee0, ee1, rowg, colg):
    f = pl.kernel(
        _aggr_body,
        out_type=[jax.ShapeDtypeStruct((_NPAD, _H), F32),
                  jax.ShapeDtypeStruct((_NPAD, _H), F32)],
        mesh=_mesh(),
        scratch_types=[
            pltpu.VMEM((2, _K * _CH), I32),
            pltpu.VMEM((_S * _CH, _H), F32),
            pltpu.VMEM_SHARED((_NPAD, _H), F32),
            pltpu.SemaphoreType.DMA, pltpu.SemaphoreType.DMA,
            pltpu.SemaphoreType.DMA, pltpu.SemaphoreType.DMA,
            pltpu.SemaphoreType.DMA, pltpu.SemaphoreType.DMA,
            pltpu.SemaphoreType.DMA, pltpu.SemaphoreType.DMA,
            pltpu.SemaphoreType.DMA, pltpu.SemaphoreType.DMA,
            pltpu.SemaphoreType.DMA, pltpu.SemaphoreType.DMA,
        ],
        compiler_params=pltpu.CompilerParams(needs_layout_passes=False),
    )
    return f(hx0, hx1, ee0, ee1, rowg, colg)


def kernel(x, edge_index, edge_attr, batch, W_ne, b_ne, W_lin, b_lin,
           root_emb, W_ee, b_ee, bn_gamma, bn_beta, bn_mean, bn_var):
    row = edge_index[0]
    col = edge_index[1]
    npad = _EPAD - _E
    rowc = jnp.concatenate([row, jnp.full((npad,), _N, I32)])
    rowg = jnp.concatenate([row, jnp.zeros((npad,), I32)])
    trash = _N + (jnp.arange(npad, dtype=I32) % (_NPAD - _N))
    colg = jnp.concatenate([col, trash])
    ea_pad = jnp.concatenate([edge_attr, jnp.zeros((npad, _DE), F32)], axis=0)

    srow, dis_full, invd_full = _degnorm(rowc, rowg)
    invd = invd_full.reshape(_NPAD, 1)
    disn = dis_full.reshape(_NPAD, 1)
    srow = srow.reshape(_EPAD, 1)

    h = _mm_bias(x, W_ne.T, b_ne.reshape(1, _D), 1000)
    for l in range(_L):
        hx0, hx1, hs0, hs1 = _mm_split4(h, W_lin[l].T, b_lin[l].reshape(1, _D),
                                        disn[:_N], 1000)
        ee0, ee1 = _mm_split_scale(ea_pad, W_ee[l].T, b_ee[l].reshape(1, _D),
                                   srow, 2048)
        ag0, ag1 = _aggr(hs0, hs1, ee0, ee1, rowg, colg)
        h = _post(ag0, ag1, hx0, hx1, invd, disn,
                  root_emb[l].reshape(1, _D), bn_gamma[l].reshape(1, _D),
                  bn_beta[l].reshape(1, _D), bn_mean[l].reshape(1, _D),
                  bn_var[l].reshape(1, _D), relu_out=(l < _L - 1), bn=1000)
    return h


# speedup vs baseline: 1.0273x; 1.0273x over previous
import functools

import jax
import jax.numpy as jnp
from jax import lax
from jax.experimental import pallas as pl
from jax.experimental.pallas import tpu as pltpu
from jax.experimental.pallas import tpu_sc as plsc

F32 = jnp.float32
I32 = jnp.int32

_N = 10000
_E = 160000
_D = 256
_DE = 16
_L = 3
_H = 128
_NPAD = 10240
_EPAD = 163840
_CH = 32
_DCH = 128
_EPS = _EPAD // 16
_NCHUNK = _EPS // _CH
_STRIPE = _NPAD // 16


def _mm_bias_body(x_ref, w_ref, b_ref, o_ref):
    o_ref[...] = jnp.dot(x_ref[...], w_ref[...],
                         preferred_element_type=F32) + b_ref[...]


def _mm_bias(x, w, b, bn):
    n, k = x.shape
    m = w.shape[1]
    return pl.pallas_call(
        _mm_bias_body,
        grid=(n // bn,),
        in_specs=[pl.BlockSpec((bn, k), lambda i: (i, 0)),
                  pl.BlockSpec((k, m), lambda i: (0, 0)),
                  pl.BlockSpec((1, m), lambda i: (0, 0))],
        out_specs=pl.BlockSpec((bn, m), lambda i: (i, 0)),
        out_shape=jax.ShapeDtypeStruct((n, m), F32),
    )(x, w, b)


def _mm_split_scale_body(x_ref, w_ref, b_ref, sc_ref, o0_ref, o1_ref):
    z = (jnp.dot(x_ref[...], w_ref[...], preferred_element_type=F32)
         + b_ref[...]) * sc_ref[...]
    o0_ref[...] = z[:, :_H]
    o1_ref[...] = z[:, _H:]


def _mm_split_scale(x, w, b, sc, bn):
    n, k = x.shape
    return pl.pallas_call(
        _mm_split_scale_body,
        grid=(n // bn,),
        in_specs=[pl.BlockSpec((bn, k), lambda i: (i, 0)),
                  pl.BlockSpec((k, _D), lambda i: (0, 0)),
                  pl.BlockSpec((1, _D), lambda i: (0, 0)),
                  pl.BlockSpec((bn, 1), lambda i: (i, 0))],
        out_specs=[pl.BlockSpec((bn, _H), lambda i: (i, 0)),
                   pl.BlockSpec((bn, _H), lambda i: (i, 0))],
        out_shape=[jax.ShapeDtypeStruct((n, _H), F32),
                   jax.ShapeDtypeStruct((n, _H), F32)],
    )(x, w, b, sc)


def _mm_split4_body(x_ref, w_ref, b_ref, sc_ref, o0_ref, o1_ref,
                    s0_ref, s1_ref):
    z = jnp.dot(x_ref[...], w_ref[...], preferred_element_type=F32) + b_ref[...]
    o0_ref[...] = z[:, :_H]
    o1_ref[...] = z[:, _H:]
    zs = z * sc_ref[...]
    s0_ref[...] = zs[:, :_H]
    s1_ref[...] = zs[:, _H:]


def _mm_split4(x, w, b, sc, bn):
    n, k = x.shape
    half = pl.BlockSpec((bn, _H), lambda i: (i, 0))
    return pl.pallas_call(
        _mm_split4_body,
        grid=(n // bn,),
        in_specs=[pl.BlockSpec((bn, k), lambda i: (i, 0)),
                  pl.BlockSpec((k, _D), lambda i: (0, 0)),
                  pl.BlockSpec((1, _D), lambda i: (0, 0)),
                  pl.BlockSpec((bn, 1), lambda i: (i, 0))],
        out_specs=[half, half, half, half],
        out_shape=[jax.ShapeDtypeStruct((n, _H), F32)] * 4,
    )(x, w, b, sc)


def _post_body(a0, a1, h0, h1, invd, disn, root, gam, bet, mu, var, o_ref, *,
               relu_out):
    iv = invd[...]
    dv = disn[...]
    for half, (a, hh) in enumerate(((a0, h0), (a1, h1))):
        sl = slice(half * _H, (half + 1) * _H)
        t = a[...] * dv + jnp.maximum(hh[...] + root[:, sl], 0.0) * iv
        t = (t - mu[:, sl]) / jnp.sqrt(var[:, sl] + 1e-5) * gam[:, sl] \
            + bet[:, sl]
        if relu_out:
            t = jnp.maximum(t, 0.0)
        o_ref[:, sl] = t


def _post(a0, a1, h0, h1, invd, disn, root, gam, bet, mu, var, relu_out, bn):
    n = _N
    body = functools.partial(_post_body, relu_out=relu_out)
    half_spec = pl.BlockSpec((bn, _H), lambda i: (i, 0))
    col_spec = pl.BlockSpec((bn, 1), lambda i: (i, 0))
    par_spec = pl.BlockSpec((1, _D), lambda i: (0, 0))
    return pl.pallas_call(
        body,
        grid=(n // bn,),
        in_specs=[half_spec, half_spec, half_spec, half_spec,
                  col_spec, col_spec,
                  par_spec, par_spec, par_spec, par_spec, par_spec],
        out_specs=pl.BlockSpec((bn, _D), lambda i: (i, 0)),
        out_shape=jax.ShapeDtypeStruct((n, _D), F32),
    )(a0, a1, h0, h1, invd, disn, root, gam, bet, mu, var)


def _mesh():
    return plsc.VectorSubcoreMesh(core_axis_name="c", subcore_axis_name="s",
                                  num_cores=2, num_subcores=16)


def _degnorm_body(rowc_hbm, rowg_hbm, srow_hbm, dis_hbm, invd_hbm,
                  idxv, onesv, rowv, degv, disv, srowv, invv, deg_sp):
    c = lax.axis_index("c")
    s = lax.axis_index("s")

    def fill_inv(k, _):
        invv[pl.ds(k * 16, 16)] = jnp.full((16,), 1.0, F32)
        return 0
    lax.fori_loop(0, 40, fill_inv, 0)
    pltpu.sync_copy(invv, deg_sp.at[pl.ds(s * 640, 640)])

    def fill_ones(k, _):
        onesv[pl.ds(k * 16, 16)] = jnp.full((16,), 1.0, F32)
        return 0
    lax.fori_loop(0, 8, fill_ones, 0)
    plsc.subcore_barrier()

    def cnt(j, _):
        base = s * _EPS + j * _DCH
        pltpu.sync_copy(rowc_hbm.at[pl.ds(base, _DCH)], idxv)
        pltpu.sync_copy(onesv, deg_sp.at[idxv], add=True)
        return 0
    lax.fori_loop(0, _EPS // _DCH, cnt, 0)
    plsc.subcore_barrier()

    pltpu.sync_copy(deg_sp, degv)
    magic = jnp.full((16,), 0x5F3759DF, I32)

    def rsq(k, _):
        d = degv[pl.ds(k * 16, 16)]
        i = plsc.bitcast(d, I32)
        y = plsc.bitcast(magic - (i >> 1), F32)
        hd = 0.5 * d
        y = y * (1.5 - hd * y * y)
        y = y * (1.5 - hd * y * y)
        y = y * (1.5 - hd * y * y)
        disv[pl.ds(k * 16, 16)] = y
        return 0
    lax.fori_loop(0, _NPAD // 16, rsq, 0)

    @pl.when(c == 0)
    def _():
        pltpu.sync_copy(disv.at[pl.ds(s * 640, 640)],
                        dis_hbm.at[pl.ds(s * 640, 640)])

    @pl.when(c == 1)
    def _():
        def invf(k, _):
            d = degv[pl.ds(s * 640 + k * 16, 16)]
            invv[pl.ds(k * 16, 16)] = 1.0 / d
            return 0
        lax.fori_loop(0, 40, invf, 0)
        pltpu.sync_copy(invv, invd_hbm.at[pl.ds(s * 640, 640)])

    half = _EPS // 2
    ebase = s * _EPS + c * half
    pltpu.sync_copy(rowg_hbm.at[pl.ds(ebase, half)], rowv)

    def srw(k, _):
        ri = rowv[pl.ds(k * 16, 16)]
        srowv[pl.ds(k * 16, 16)] = plsc.load_gather(disv, [ri])
        return 0
    lax.fori_loop(0, half // 16, srw, 0)
    pltpu.sync_copy(srowv, srow_hbm.at[pl.ds(ebase, half)])


def _degnorm(rowc, rowg):
    f = pl.kernel(
        _degnorm_body,
        out_type=[jax.ShapeDtypeStruct((_EPAD,), F32),
                  jax.ShapeDtypeStruct((_NPAD,), F32),
                  jax.ShapeDtypeStruct((_NPAD,), F32)],
        mesh=_mesh(),
        scratch_types=[
            pltpu.VMEM((_DCH,), I32),
            pltpu.VMEM((_DCH,), F32),
            pltpu.VMEM((_EPS // 2,), I32),
            pltpu.VMEM((_NPAD,), F32),
            pltpu.VMEM((_NPAD,), F32),
            pltpu.VMEM((_EPS // 2,), F32),
            pltpu.VMEM((640,), F32),
            pltpu.VMEM_SHARED((_NPAD,), F32),
        ],
        compiler_params=pltpu.CompilerParams(needs_layout_passes=False),
    )
    return f(rowc, rowg)


_K = 16
_S = 8


def _aggr_body(hx0, hx1, ee0, ee1, rowg, colg, out0, out1,
               idxb, gb, agg_sp, *sems):
    c = lax.axis_index("c")
    s = lax.axis_index("s")
    sem_g = sems[:_S]
    sem_e = sems[_S:2 * _S]
    sem_s = sems[2 * _S:]

    def z(t, _):
        gb[t // 8, pl.ds((t % 8) * 16, 16)] = jnp.zeros((16,), F32)
        return 0
    lax.fori_loop(0, _S * _CH * 8, z, 0)
    nz = _S * _CH
    for q in range(0, _STRIPE, nz):
        rows = min(nz, _STRIPE - q)
        pltpu.sync_copy(gb.at[pl.ds(0, rows)],
                        agg_sp.at[pl.ds(s * _STRIPE + q, rows)])
    plsc.subcore_barrier()

    def run(hx, ee, out):
        def body(t, _):
            base = s * _EPS + t * (_K * _CH)
            pltpu.sync_copy(rowg.at[pl.ds(base, _K * _CH)], idxb.at[0])
            pltpu.sync_copy(colg.at[pl.ds(base, _K * _CH)], idxb.at[1])
            cg, ce, sc = {}, {}, {}

            def issue_ee(k):
                es = k % _S
                ce[k] = pltpu.async_copy(
                    ee.at[pl.ds(base + k * _CH, _CH)],
                    gb.at[pl.ds(es * _CH, _CH)], sem_e[es])

            def issue_gather(k):
                gs = k % _S
                cg[k] = pltpu.async_copy(
                    hx.at[idxb.at[0, pl.ds(k * _CH, _CH)]],
                    gb.at[pl.ds(gs * _CH, _CH)], sem_g[gs], add=True)

            for k in range(_S):
                issue_ee(k)
            ce[0].wait()
            issue_gather(0)
            for k in range(_K):
                gs = k % _S
                go = gs * _CH
                if k + 1 < _K:
                    ce[k + 1].wait()
                    issue_gather(k + 1)
                cg[k].wait()

                def group(g, _, go=go):
                    for j in range(8):
                        i = g * 8 + j
                        for r in range(_H // 16):
                            v = gb[go + i, pl.ds(r * 16, 16)]
                            gb[go + i, pl.ds(r * 16, 16)] = \
                                jnp.maximum(v, 0.0)
                    return 0
                lax.fori_loop(0, _CH // 8, group, 0)

                sc[k] = pltpu.async_copy(
                    gb.at[pl.ds(go, _CH)],
                    agg_sp.at[idxb.at[1, pl.ds(k * _CH, _CH)]], sem_s[gs],
                    add=True)
                if k + _S < _K:
                    sc[k].wait()
                    issue_ee(k + _S)
            for k in range(_K - _S, _K):
                sc[k].wait()
            return 0
        lax.fori_loop(0, _NCHUNK // _K, body, 0)
        plsc.subcore_barrier()
        pltpu.sync_copy(agg_sp.at[pl.ds(s * _STRIPE, _STRIPE)],
                        out.at[pl.ds(s * _STRIPE, _STRIPE)])

    @pl.when(c == 0)
    def _():
        run(hx0, ee0, out0)

    @pl.when(c == 1)
    def _():
        run(hx1, ee1, out1)


def _aggr(hx0, hx1, ee0, ee1, rowg, colg):
    f = pl.kernel(
        _aggr_body,
        out_type=[jax.ShapeDtypeStruct((_NPAD, _H), F32),
                  jax.ShapeDtypeStruct((_NPAD, _H), F32)],
        mesh=_mesh(),
        scratch_types=[
            pltpu.VMEM((2, _K * _CH), I32),
            pltpu.VMEM((_S * _CH, _H), F32),
            pltpu.VMEM_SHARED((_NPAD, _H), F32),
        ] + [pltpu.SemaphoreType.DMA] * (3 * _S),
        compiler_params=pltpu.CompilerParams(needs_layout_passes=False),
    )
    return f(hx0, hx1, ee0, ee1, rowg, colg)


def kernel(x, edge_index, edge_attr, batch, W_ne, b_ne, W_lin, b_lin,
           root_emb, W_ee, b_ee, bn_gamma, bn_beta, bn_mean, bn_var):
    row = edge_index[0]
    col = edge_index[1]
    npad = _EPAD - _E
    rowc = jnp.concatenate([row, jnp.full((npad,), _N, I32)])
    rowg = jnp.concatenate([row, jnp.zeros((npad,), I32)])
    trash = _N + (jnp.arange(npad, dtype=I32) % (_NPAD - _N))
    colg = jnp.concatenate([col, trash])
    ea_pad = jnp.concatenate([edge_attr, jnp.zeros((npad, _DE), F32)], axis=0)

    srow, dis_full, invd_full = _degnorm(rowc, rowg)
    invd = invd_full.reshape(_NPAD, 1)
    disn = dis_full.reshape(_NPAD, 1)
    srow = srow.reshape(_EPAD, 1)

    h = _mm_bias(x, W_ne.T, b_ne.reshape(1, _D), 1000)
    for l in range(_L):
        hx0, hx1, hs0, hs1 = _mm_split4(h, W_lin[l].T, b_lin[l].reshape(1, _D),
                                        disn[:_N], 1000)
        ee0, ee1 = _mm_split_scale(ea_pad, W_ee[l].T, b_ee[l].reshape(1, _D),
                                   srow, 2048)
        ag0, ag1 = _aggr(hs0, hs1, ee0, ee1, rowg, colg)
        h = _post(ag0, ag1, hx0, hx1, invd, disn,
                  root_emb[l].reshape(1, _D), bn_gamma[l].reshape(1, _D),
                  bn_beta[l].reshape(1, _D), bn_mean[l].reshape(1, _D),
                  bn_var[l].reshape(1, _D), relu_out=(l < _L - 1), bn=1000)
    return h


# speedup vs baseline: 1.0494x; 1.0215x over previous
import functools

import jax
import jax.numpy as jnp
from jax import lax
from jax.experimental import pallas as pl
from jax.experimental.pallas import tpu as pltpu
from jax.experimental.pallas import tpu_sc as plsc

F32 = jnp.float32
I32 = jnp.int32

_N = 10000
_E = 160000
_D = 256
_DE = 16
_L = 3
_H = 128
_NPAD = 10240
_EPAD = 163840
_CH = 64
_DCH = 128
_EPS = _EPAD // 16
_NCHUNK = _EPS // _CH
_STRIPE = _NPAD // 16


def _mm_bias_body(x_ref, w_ref, b_ref, o_ref):
    o_ref[...] = jnp.dot(x_ref[...], w_ref[...],
                         preferred_element_type=F32) + b_ref[...]


def _mm_bias(x, w, b, bn):
    n, k = x.shape
    m = w.shape[1]
    return pl.pallas_call(
        _mm_bias_body,
        grid=(n // bn,),
        in_specs=[pl.BlockSpec((bn, k), lambda i: (i, 0)),
                  pl.BlockSpec((k, m), lambda i: (0, 0)),
                  pl.BlockSpec((1, m), lambda i: (0, 0))],
        out_specs=pl.BlockSpec((bn, m), lambda i: (i, 0)),
        out_shape=jax.ShapeDtypeStruct((n, m), F32),
    )(x, w, b)


def _mm_split_scale_body(x_ref, w_ref, b_ref, sc_ref, o0_ref, o1_ref):
    z = (jnp.dot(x_ref[...], w_ref[...], preferred_element_type=F32)
         + b_ref[...]) * sc_ref[...]
    o0_ref[...] = z[:, :_H]
    o1_ref[...] = z[:, _H:]


def _mm_split_scale(x, w, b, sc, bn):
    n, k = x.shape
    return pl.pallas_call(
        _mm_split_scale_body,
        grid=(n // bn,),
        in_specs=[pl.BlockSpec((bn, k), lambda i: (i, 0)),
                  pl.BlockSpec((k, _D), lambda i: (0, 0)),
                  pl.BlockSpec((1, _D), lambda i: (0, 0)),
                  pl.BlockSpec((bn, 1), lambda i: (i, 0))],
        out_specs=[pl.BlockSpec((bn, _H), lambda i: (i, 0)),
                   pl.BlockSpec((bn, _H), lambda i: (i, 0))],
        out_shape=[jax.ShapeDtypeStruct((n, _H), F32),
                   jax.ShapeDtypeStruct((n, _H), F32)],
    )(x, w, b, sc)


def _mm_split4_body(x_ref, w_ref, b_ref, sc_ref, o0_ref, o1_ref,
                    s0_ref, s1_ref):
    z = jnp.dot(x_ref[...], w_ref[...], preferred_element_type=F32) + b_ref[...]
    o0_ref[...] = z[:, :_H]
    o1_ref[...] = z[:, _H:]
    zs = z * sc_ref[...]
    s0_ref[...] = zs[:, :_H]
    s1_ref[...] = zs[:, _H:]


def _mm_split4(x, w, b, sc, bn):
    n, k = x.shape
    half = pl.BlockSpec((bn, _H), lambda i: (i, 0))
    return pl.pallas_call(
        _mm_split4_body,
        grid=(n // bn,),
        in_specs=[pl.BlockSpec((bn, k), lambda i: (i, 0)),
                  pl.BlockSpec((k, _D), lambda i: (0, 0)),
                  pl.BlockSpec((1, _D), lambda i: (0, 0)),
                  pl.BlockSpec((bn, 1), lambda i: (i, 0))],
        out_specs=[half, half, half, half],
        out_shape=[jax.ShapeDtypeStruct((n, _H), F32)] * 4,
    )(x, w, b, sc)


def _post_body(a0, a1, h0, h1, invd, disn, root, gam, bet, mu, var, o_ref, *,
               relu_out):
    iv = invd[...]
    dv = disn[...]
    for half, (a, hh) in enumerate(((a0, h0), (a1, h1))):
        sl = slice(half * _H, (half + 1) * _H)
        t = a[...] * dv + jnp.maximum(hh[...] + root[:, sl], 0.0) * iv
        t = (t - mu[:, sl]) / jnp.sqrt(var[:, sl] + 1e-5) * gam[:, sl] \
            + bet[:, sl]
        if relu_out:
            t = jnp.maximum(t, 0.0)
        o_ref[:, sl] = t


def _post(a0, a1, h0, h1, invd, disn, root, gam, bet, mu, var, relu_out, bn):
    n = _N
    body = functools.partial(_post_body, relu_out=relu_out)
    half_spec = pl.BlockSpec((bn, _H), lambda i: (i, 0))
    col_spec = pl.BlockSpec((bn, 1), lambda i: (i, 0))
    par_spec = pl.BlockSpec((1, _D), lambda i: (0, 0))
    return pl.pallas_call(
        body,
        grid=(n // bn,),
        in_specs=[half_spec, half_spec, half_spec, half_spec,
                  col_spec, col_spec,
                  par_spec, par_spec, par_spec, par_spec, par_spec],
        out_specs=pl.BlockSpec((bn, _D), lambda i: (i, 0)),
        out_shape=jax.ShapeDtypeStruct((n, _D), F32),
    )(a0, a1, h0, h1, invd, disn, root, gam, bet, mu, var)


def _mesh():
    return plsc.VectorSubcoreMesh(core_axis_name="c", subcore_axis_name="s",
                                  num_cores=2, num_subcores=16)


def _degnorm_body(rowc_hbm, rowg_hbm, srow_hbm, dis_hbm, invd_hbm,
                  idxv, onesv, rowv, degv, disv, srowv, invv, deg_sp):
    c = lax.axis_index("c")
    s = lax.axis_index("s")

    def fill_inv(k, _):
        invv[pl.ds(k * 16, 16)] = jnp.full((16,), 1.0, F32)
        return 0
    lax.fori_loop(0, 40, fill_inv, 0)
    pltpu.sync_copy(invv, deg_sp.at[pl.ds(s * 640, 640)])

    def fill_ones(k, _):
        onesv[pl.ds(k * 16, 16)] = jnp.full((16,), 1.0, F32)
        return 0
    lax.fori_loop(0, 8, fill_ones, 0)
    plsc.subcore_barrier()

    def cnt(j, _):
        base = s * _EPS + j * _DCH
        pltpu.sync_copy(rowc_hbm.at[pl.ds(base, _DCH)], idxv)
        pltpu.sync_copy(onesv, deg_sp.at[idxv], add=True)
        return 0
    lax.fori_loop(0, _EPS // _DCH, cnt, 0)
    plsc.subcore_barrier()

    pltpu.sync_copy(deg_sp, degv)
    magic = jnp.full((16,), 0x5F3759DF, I32)

    def rsq(k, _):
        d = degv[pl.ds(k * 16, 16)]
        i = plsc.bitcast(d, I32)
        y = plsc.bitcast(magic - (i >> 1), F32)
        hd = 0.5 * d
        y = y * (1.5 - hd * y * y)
        y = y * (1.5 - hd * y * y)
        y = y * (1.5 - hd * y * y)
        disv[pl.ds(k * 16, 16)] = y
        return 0
    lax.fori_loop(0, _NPAD // 16, rsq, 0)

    @pl.when(c == 0)
    def _():
        pltpu.sync_copy(disv.at[pl.ds(s * 640, 640)],
                        dis_hbm.at[pl.ds(s * 640, 640)])

    @pl.when(c == 1)
    def _():
        def invf(k, _):
            d = degv[pl.ds(s * 640 + k * 16, 16)]
            invv[pl.ds(k * 16, 16)] = 1.0 / d
            return 0
        lax.fori_loop(0, 40, invf, 0)
        pltpu.sync_copy(invv, invd_hbm.at[pl.ds(s * 640, 640)])

    half = _EPS // 2
    ebase = s * _EPS + c * half
    pltpu.sync_copy(rowg_hbm.at[pl.ds(ebase, half)], rowv)

    def srw(k, _):
        ri = rowv[pl.ds(k * 16, 16)]
        srowv[pl.ds(k * 16, 16)] = plsc.load_gather(disv, [ri])
        return 0
    lax.fori_loop(0, half // 16, srw, 0)
    pltpu.sync_copy(srowv, srow_hbm.at[pl.ds(ebase, half)])


def _degnorm(rowc, rowg):
    f = pl.kernel(
        _degnorm_body,
        out_type=[jax.ShapeDtypeStruct((_EPAD,), F32),
                  jax.ShapeDtypeStruct((_NPAD,), F32),
                  jax.ShapeDtypeStruct((_NPAD,), F32)],
        mesh=_mesh(),
        scratch_types=[
            pltpu.VMEM((_DCH,), I32),
            pltpu.VMEM((_DCH,), F32),
            pltpu.VMEM((_EPS // 2,), I32),
            pltpu.VMEM((_NPAD,), F32),
            pltpu.VMEM((_NPAD,), F32),
            pltpu.VMEM((_EPS // 2,), F32),
            pltpu.VMEM((640,), F32),
            pltpu.VMEM_SHARED((_NPAD,), F32),
        ],
        compiler_params=pltpu.CompilerParams(needs_layout_passes=False),
    )
    return f(rowc, rowg)


_K = 8
_S = 4


def _aggr_body(hx0, hx1, ee0, ee1, rowg, colg, out0, out1,
               idxb, gb, agg_sp, *sems):
    c = lax.axis_index("c")
    s = lax.axis_index("s")
    sem_g = sems[:_S]
    sem_e = sems[_S:2 * _S]
    sem_s = sems[2 * _S:]

    def z(t, _):
        gb[t // 8, pl.ds((t % 8) * 16, 16)] = jnp.zeros((16,), F32)
        return 0
    lax.fori_loop(0, _S * _CH * 8, z, 0)
    nz = _S * _CH
    for q in range(0, _STRIPE, nz):
        rows = min(nz, _STRIPE - q)
        pltpu.sync_copy(gb.at[pl.ds(0, rows)],
                        agg_sp.at[pl.ds(s * _STRIPE + q, rows)])
    plsc.subcore_barrier()

    def run(hx, ee, out):
        def body(t, _):
            base = s * _EPS + t * (_K * _CH)
            pltpu.sync_copy(rowg.at[pl.ds(base, _K * _CH)], idxb.at[0])
            pltpu.sync_copy(colg.at[pl.ds(base, _K * _CH)], idxb.at[1])
            cg, ce, sc = {}, {}, {}

            def issue_ee(k):
                es = k % _S
                ce[k] = pltpu.async_copy(
                    ee.at[pl.ds(base + k * _CH, _CH)],
                    gb.at[pl.ds(es * _CH, _CH)], sem_e[es])

            def issue_gather(k):
                gs = k % _S
                cg[k] = pltpu.async_copy(
                    hx.at[idxb.at[0, pl.ds(k * _CH, _CH)]],
                    gb.at[pl.ds(gs * _CH, _CH)], sem_g[gs], add=True)

            for k in range(_S):
                issue_ee(k)
            ce[0].wait()
            issue_gather(0)
            for k in range(_K):
                gs = k % _S
                go = gs * _CH
                if k + 1 < _K:
                    ce[k + 1].wait()
                    issue_gather(k + 1)
                cg[k].wait()

                def group(g, _, go=go):
                    for j in range(8):
                        i = g * 8 + j
                        for r in range(_H // 16):
                            v = gb[go + i, pl.ds(r * 16, 16)]
                            gb[go + i, pl.ds(r * 16, 16)] = \
                                jnp.maximum(v, 0.0)
                    return 0
                lax.fori_loop(0, _CH // 8, group, 0)

                sc[k] = pltpu.async_copy(
                    gb.at[pl.ds(go, _CH)],
                    agg_sp.at[idxb.at[1, pl.ds(k * _CH, _CH)]], sem_s[gs],
                    add=True)
                if k + _S < _K:
                    sc[k].wait()
                    issue_ee(k + _S)
            for k in range(_K - _S, _K):
                sc[k].wait()
            return 0
        lax.fori_loop(0, _NCHUNK // _K, body, 0)
        plsc.subcore_barrier()
        pltpu.sync_copy(agg_sp.at[pl.ds(s * _STRIPE, _STRIPE)],
                        out.at[pl.ds(s * _STRIPE, _STRIPE)])

    @pl.when(c == 0)
    def _():
        run(hx0, ee0, out0)

    @pl.when(c == 1)
    def _():
        run(hx1, ee1, out1)


def _aggr(hx0, hx1, ee0, ee1, rowg, colg):
    f = pl.kernel(
        _aggr_body,
        out_type=[jax.ShapeDtypeStruct((_NPAD, _H), F32),
                  jax.ShapeDtypeStruct((_NPAD, _H), F32)],
        mesh=_mesh(),
        scratch_types=[
            pltpu.VMEM((2, _K * _CH), I32),
            pltpu.VMEM((_S * _CH, _H), F32),
            pltpu.VMEM_SHARED((_NPAD, _H), F32),
        ] + [pltpu.SemaphoreType.DMA] * (3 * _S),
        compiler_params=pltpu.CompilerParams(needs_layout_passes=False),
    )
    return f(hx0, hx1, ee0, ee1, rowg, colg)


def kernel(x, edge_index, edge_attr, batch, W_ne, b_ne, W_lin, b_lin,
           root_emb, W_ee, b_ee, bn_gamma, bn_beta, bn_mean, bn_var):
    row = edge_index[0]
    col = edge_index[1]
    npad = _EPAD - _E
    rowc = jnp.concatenate([row, jnp.full((npad,), _N, I32)])
    rowg = jnp.concatenate([row, jnp.zeros((npad,), I32)])
    trash = _N + (jnp.arange(npad, dtype=I32) % (_NPAD - _N))
    colg = jnp.concatenate([col, trash])
    ea_pad = jnp.concatenate([edge_attr, jnp.zeros((npad, _DE), F32)], axis=0)

    srow, dis_full, invd_full = _degnorm(rowc, rowg)
    invd = invd_full.reshape(_NPAD, 1)
    disn = dis_full.reshape(_NPAD, 1)
    srow = srow.reshape(_EPAD, 1)

    h = _mm_bias(x, W_ne.T, b_ne.reshape(1, _D), 1000)
    for l in range(_L):
        hx0, hx1, hs0, hs1 = _mm_split4(h, W_lin[l].T, b_lin[l].reshape(1, _D),
                                        disn[:_N], 1000)
        ee0, ee1 = _mm_split_scale(ea_pad, W_ee[l].T, b_ee[l].reshape(1, _D),
                                   srow, 2048)
        ag0, ag1 = _aggr(hs0, hs1, ee0, ee1, rowg, colg)
        h = _post(ag0, ag1, hx0, hx1, invd, disn,
                  root_emb[l].reshape(1, _D), bn_gamma[l].reshape(1, _D),
                  bn_beta[l].reshape(1, _D), bn_mean[l].reshape(1, _D),
                  bn_var[l].reshape(1, _D), relu_out=(l < _L - 1), bn=1000)
    return h
